# baseline (speedup 1.0000x reference)
import functools

import jax
import jax.numpy as jnp
from jax import lax
from jax.experimental import pallas as pl
from jax.experimental.pallas import tpu as pltpu
from jax.experimental.pallas import tpu_sc as plsc

B = 1024
L = 200
WORD_DIM = 128
POS_DIM = 16
OUT_D = WORD_DIM + 2 * POS_DIM

NC = 2
NS = 16
NW = NC * NS
BT = B // 128
CT = OUT_D // 8
NITEM = L * BT
IPW = NITEM // NW
NB = 2


def _lookup(word_table, pos1_table, pos2_table, wi, p1i, p2i):
    mesh = plsc.VectorSubcoreMesh(
        core_axis_name="c", subcore_axis_name="s", num_cores=NC, num_subcores=NS
    )

    @functools.partial(
        pl.kernel,
        out_type=jax.ShapeDtypeStruct((L, CT, BT, 8, 128), jnp.float32),
        mesh=mesh,
        compiler_params=pltpu.CompilerParams(
            use_tc_tiling_on_sc=False, needs_layout_passes=False),
        scratch_types=[
            pltpu.VMEM((IPW, 128), jnp.int32),
            pltpu.VMEM((IPW, 128), jnp.int32),
            pltpu.VMEM((IPW, 128), jnp.int32),
            pltpu.VMEM((NB, 128, WORD_DIM), jnp.float32),
            pltpu.VMEM((NB, 128, POS_DIM), jnp.float32),
            pltpu.VMEM((NB, 128, POS_DIM), jnp.float32),
            pltpu.VMEM((NB, CT, 8, 128), jnp.float32),
        ]
        + [pltpu.SemaphoreType.DMA] * NB
        + [pltpu.SemaphoreType.DMA] * NB,
    )
    def k(wt, p1t, p2t, wi_h, p1i_h, p2i_h, out,
          widx_v, p1idx_v, p2idx_v, w_v, p1_v, p2_v, o_v, *sems):
        sem_g = sems[:NB]
        sem_s = sems[NB:]
        wid = lax.axis_index("s") * NC + lax.axis_index("c")
        t0g = wid * IPW
        pltpu.sync_copy(wi_h.at[wid], widx_v)
        pltpu.sync_copy(p1i_h.at[wid], p1idx_v)
        pltpu.sync_copy(p2i_h.at[wid], p2idx_v)

        lane = lax.iota(jnp.int32, 16)

        def gather_copies(j, b, issue):
            f = pltpu.async_copy if issue else (
                lambda s, d, m: pltpu.make_async_copy(s, d, m).wait())
            f(wt.at[widx_v.at[j]], w_v.at[b], sem_g[b])
            f(p1t.at[p1idx_v.at[j]], p1_v.at[b], sem_g[b])
            f(p2t.at[p2idx_v.at[j]], p2_v.at[b], sem_g[b])

        def scatter_copy(j, b, issue):
            t = t0g + j
            l = t // BT
            cb = t % BT
            dst = out.at[l, pl.ds(0, CT), cb]
            f = pltpu.async_copy if issue else (
                lambda s, d, m: pltpu.make_async_copy(s, d, m).wait())
            f(o_v.at[b], dst, sem_s[b])

        def transpose_item(b):
            @plsc.parallel_loop(0, WORD_DIM // 2, unroll=8)
            def _(k):
                cols = jnp.full((16,), k, jnp.int32)
                for r0 in range(8):
                    v = plsc.load_gather(w_v.at[b], [r0 * 16 + lane, cols])
                    o_v[b, k // 8, k % 8, pl.ds(r0 * 16, 16)] = v
                for dr in range(2):
                    r = 2 * k + dr
                    rv = jnp.full((16,), r, jnp.int32)
                    for c0 in (64, 80, 96, 112):
                        ct_v = (c0 + lane) // 8
                        t2_v = (c0 + lane) % 8
                        v = w_v[b, r, pl.ds(c0, 16)]
                        plsc.store_scatter(o_v.at[b], [ct_v, t2_v, rv], v)

            for src, ct_base in ((p1_v, 16), (p2_v, 18)):
                @plsc.parallel_loop(0, POS_DIM, unroll=8)
                def _(kk):
                    cols = jnp.full((16,), kk, jnp.int32)
                    for r0 in range(8):
                        rows = r0 * 16 + lane
                        v = plsc.load_gather(src.at[b], [rows, cols])
                        o_v[b, ct_base + kk // 8, kk % 8, pl.ds(r0 * 16, 16)] = v

        for b in range(NB):
            gather_copies(b, b, True)

        def step(i, carry):
            for b in range(NB):
                j = i * NB + b
                gather_copies(j, b, False)
                if b == 0:
                    @pl.when(i >= 1)
                    def _():
                        scatter_copy(j - NB, b, False)
                else:
                    @pl.when(i >= 1)
                    def _():
                        scatter_copy(j - NB, b, False)
                transpose_item(b)
                @pl.when(j + NB < IPW)
                def _():
                    gather_copies(j + NB, b, True)
                scatter_copy(j, b, True)
            return carry

        lax.fori_loop(0, IPW // NB, step, 0)
        for b in range(NB):
            scatter_copy(IPW - NB + b, b, False)

    return k(word_table, pos1_table, pos2_table, wi, p1i, p2i)


def kernel(word_table, pos1_table, pos2_table, word, pos1, pos2):
    wi = word.astype(jnp.int32).T.reshape(NW, IPW, 128)
    p1i = pos1.astype(jnp.int32).T.reshape(NW, IPW, 128)
    p2i = pos2.astype(jnp.int32).T.reshape(NW, IPW, 128)
    out5 = _lookup(word_table, pos1_table, pos2_table, wi, p1i, p2i)
    return out5.transpose(2, 4, 0, 1, 3).reshape(B, L, OUT_D)


# speedup vs baseline: 1.2022x; 1.2022x over previous
import functools

import jax
import jax.numpy as jnp
from jax import lax
from jax.experimental import pallas as pl
from jax.experimental.pallas import tpu as pltpu
from jax.experimental.pallas import tpu_sc as plsc

B = 1024
L = 200
WORD_DIM = 128
POS_DIM = 16
OUT_D = WORD_DIM + 2 * POS_DIM

NC = 2
NS = 16
NW = NC * NS
BT = B // 128
CT = OUT_D // 8
NITEM = L * BT
IPW = NITEM // NW
NB = 2


def _lookup(word_table, pos1_table, pos2_table, wi, p1i, p2i):
    mesh = plsc.VectorSubcoreMesh(
        core_axis_name="c", subcore_axis_name="s", num_cores=NC, num_subcores=NS
    )

    @functools.partial(
        pl.kernel,
        out_type=jax.ShapeDtypeStruct((L, CT, BT, 8, 128), jnp.float32),
        mesh=mesh,
        compiler_params=pltpu.CompilerParams(
            use_tc_tiling_on_sc=False, needs_layout_passes=False),
        scratch_types=[
            pltpu.VMEM((IPW, 128), jnp.int32),
            pltpu.VMEM((IPW, 128), jnp.int32),
            pltpu.VMEM((IPW, 128), jnp.int32),
            pltpu.VMEM((NB, 128, WORD_DIM), jnp.float32),
            pltpu.VMEM((NB, 128, POS_DIM), jnp.float32),
            pltpu.VMEM((NB, 128, POS_DIM), jnp.float32),
            pltpu.VMEM((NB, CT, 8, 128), jnp.float32),
        ]
        + [pltpu.SemaphoreType.DMA] * NB
        + [pltpu.SemaphoreType.DMA] * NB,
    )
    def k(wt, p1t, p2t, wi_h, p1i_h, p2i_h, out,
          widx_v, p1idx_v, p2idx_v, w_v, p1_v, p2_v, o_v, *sems):
        sem_g = sems[:NB]
        sem_s = sems[NB:]
        wid = lax.axis_index("s") * NC + lax.axis_index("c")
        t0g = wid * IPW
        pltpu.sync_copy(wi_h.at[wid], widx_v)
        pltpu.sync_copy(p1i_h.at[wid], p1idx_v)
        pltpu.sync_copy(p2i_h.at[wid], p2idx_v)

        lane = lax.iota(jnp.int32, 16)

        def gather_copies(j, b, issue):
            f = pltpu.async_copy if issue else (
                lambda s, d, m: pltpu.make_async_copy(s, d, m).wait())
            f(wt.at[widx_v.at[j]], w_v.at[b], sem_g[b])
            f(p1t.at[p1idx_v.at[j]], p1_v.at[b], sem_g[b])
            f(p2t.at[p2idx_v.at[j]], p2_v.at[b], sem_g[b])

        def scatter_copy(j, b, issue):
            t = t0g + j
            l = t // BT
            cb = t % BT
            dst = out.at[l, pl.ds(0, CT), cb]
            f = pltpu.async_copy if issue else (
                lambda s, d, m: pltpu.make_async_copy(s, d, m).wait())
            f(o_v.at[b], dst, sem_s[b])

        def transpose_item(b):
            @plsc.parallel_loop(0, WORD_DIM // 2, unroll=4)
            def _(k):
                cols = jnp.full((16,), k, jnp.int32)
                for r0 in range(8):
                    v = plsc.load_gather(w_v.at[b], [r0 * 16 + lane, cols])
                    o_v[b, k // 8, k % 8, pl.ds(r0 * 16, 16)] = v
                for dr in range(2):
                    r = 2 * k + dr
                    rv = jnp.full((16,), r, jnp.int32)
                    for c0 in (64, 80, 96, 112):
                        ct_v = (c0 + lane) // 8
                        t2_v = (c0 + lane) % 8
                        v = w_v[b, r, pl.ds(c0, 16)]
                        plsc.store_scatter(o_v.at[b], [ct_v, t2_v, rv], v)

            for src, ct_base in ((p1_v, 16), (p2_v, 18)):
                @plsc.parallel_loop(0, POS_DIM, unroll=8)
                def _(kk):
                    cols = jnp.full((16,), kk, jnp.int32)
                    for r0 in range(8):
                        rows = r0 * 16 + lane
                        v = plsc.load_gather(src.at[b], [rows, cols])
                        o_v[b, ct_base + kk // 8, kk % 8, pl.ds(r0 * 16, 16)] = v

        for b in range(NB):
            gather_copies(b, b, True)

        def step(i, carry):
            for b in range(NB):
                j = i * NB + b
                gather_copies(j, b, False)
                if b == 0:
                    @pl.when(i >= 1)
                    def _():
                        scatter_copy(j - NB, b, False)
                else:
                    @pl.when(i >= 1)
                    def _():
                        scatter_copy(j - NB, b, False)
                transpose_item(b)
                @pl.when(j + NB < IPW)
                def _():
                    gather_copies(j + NB, b, True)
                scatter_copy(j, b, True)
            return carry

        lax.fori_loop(0, IPW // NB, step, 0)
        for b in range(NB):
            scatter_copy(IPW - NB + b, b, False)

    return k(word_table, pos1_table, pos2_table, wi, p1i, p2i)


def kernel(word_table, pos1_table, pos2_table, word, pos1, pos2):
    wi = word.astype(jnp.int32).T.reshape(NW, IPW, 128)
    p1i = pos1.astype(jnp.int32).T.reshape(NW, IPW, 128)
    p2i = pos2.astype(jnp.int32).T.reshape(NW, IPW, 128)
    out5 = _lookup(word_table, pos1_table, pos2_table, wi, p1i, p2i)
    return out5.transpose(2, 4, 0, 1, 3).reshape(B, L, OUT_D)
